# Initial kernel scaffold; baseline (speedup 1.0000x reference)
import jax
import jax.numpy as jnp
from jax.experimental import pallas as pl


def kernel(nfeats, efeats, edge_index, W1_w, W1_b, A1_w, A1_b, W2_w, W2_b, A2_w, A2_b):
    raise NotImplementedError("write your pallas kernel here")


# speedup vs baseline: 10.0238x; 10.0238x over previous
import jax
import jax.numpy as jnp
from jax import lax
from jax.experimental import pallas as pl
from jax.experimental.pallas import tpu as pltpu
from jax.experimental.pallas import tpu_sc as plsc

N = 10000
E = 320000
D = 128
DE = 16
AW = 2 * DE

NC = 2
NS = 16
NW = NC * NS
EPT = E // NW
CH = 80
NCHUNK = EPT // CH
RPT = 624
REM_OFF = NS * RPT
REM = N - REM_OFF

_EPS = 1e-12


def _edge_body(src_hbm, dst_hbm, as_hbm, ad_hbm, ef_hbm, zn_out,
               srcb, dstb, s1b, s2b, efb, msgb, zrow, zn_sh, sem):
    c = lax.axis_index("c")
    s = lax.axis_index("s")
    wid = s * NC + c

    zeros = jnp.zeros((16,), jnp.float32)

    def _z16(i, _):
        zrow[i, pl.ds(0, 16)] = zeros
        zrow[i, pl.ds(16, 16)] = zeros
        return 0

    lax.fori_loop(0, RPT, _z16, 0)

    rows = pl.ds(s * RPT, RPT)
    rem = pl.ds(REM_OFF, REM)
    pltpu.sync_copy(zrow, zn_sh.at[rows])

    @pl.when(s == NS - 1)
    def _zero_rem():
        pltpu.sync_copy(zrow.at[pl.ds(0, REM)], zn_sh.at[rem])

    plsc.subcore_barrier()

    def _chunk(k, _):
        base = wid * EPT + k * CH
        pltpu.sync_copy(src_hbm.at[pl.ds(base, CH)], srcb)
        pltpu.sync_copy(dst_hbm.at[pl.ds(base, CH)], dstb)
        pltpu.sync_copy(ef_hbm.at[pl.ds(base, CH)], efb)
        pltpu.async_copy(as_hbm.at[srcb], s1b, sem).wait()
        pltpu.async_copy(ad_hbm.at[dstb], s2b, sem).wait()

        def _edge(e, _):
            w = jnp.exp(jnp.maximum(s1b[e, :] + s2b[e, :], 0.0))
            msgb[e, pl.ds(0, DE)] = w * efb[e, :]
            msgb[e, pl.ds(DE, DE)] = w
            return 0

        lax.fori_loop(0, CH, _edge, 0)

        pltpu.sync_copy(msgb, zn_sh.at[dstb], add=True)
        return 0

    lax.fori_loop(0, NCHUNK, _chunk, 0)
    plsc.subcore_barrier()

    pltpu.sync_copy(zn_sh.at[rows], zrow)
    pltpu.sync_copy(zrow, zn_out.at[c, rows])

    @pl.when(s == NS - 1)
    def _flush_rem():
        pltpu.sync_copy(zn_sh.at[rem], zrow.at[pl.ds(0, REM)])
        pltpu.sync_copy(zrow.at[pl.ds(0, REM)], zn_out.at[c, rem])


_edge_pass = pl.kernel(
    _edge_body,
    out_type=jax.ShapeDtypeStruct((NC, N, AW), jnp.float32),
    mesh=plsc.VectorSubcoreMesh(core_axis_name="c", subcore_axis_name="s"),
    compiler_params=pltpu.CompilerParams(use_tc_tiling_on_sc=False),
    scratch_types=[
        pltpu.VMEM((CH,), jnp.int32),
        pltpu.VMEM((CH,), jnp.int32),
        pltpu.VMEM((CH, DE), jnp.float32),
        pltpu.VMEM((CH, DE), jnp.float32),
        pltpu.VMEM((CH, DE), jnp.float32),
        pltpu.VMEM((CH, AW), jnp.float32),
        pltpu.VMEM((RPT, AW), jnp.float32),
        pltpu.VMEM_SHARED((N, AW), jnp.float32),
        pltpu.SemaphoreType.DMA,
    ],
)


BLK = 2000


def _proj_body(x_ref, ap_ref, ab_ref, s_ref, d_ref):
    a = (jnp.dot(x_ref[...], ap_ref[...], preferred_element_type=jnp.float32)
         + ab_ref[...])
    s_ref[...] = jnp.broadcast_to(a[:, 0:1], (a.shape[0], DE))
    d_ref[...] = jnp.broadcast_to(a[:, 1:2], (a.shape[0], DE))


def _proj(x, apair, abias):
    return pl.pallas_call(
        _proj_body,
        grid=(N // BLK,),
        in_specs=[
            pl.BlockSpec((BLK, D), lambda i: (i, 0)),
            pl.BlockSpec((D, 2), lambda i: (0, 0)),
            pl.BlockSpec((1, 2), lambda i: (0, 0)),
        ],
        out_specs=[
            pl.BlockSpec((BLK, DE), lambda i: (i, 0)),
            pl.BlockSpec((BLK, DE), lambda i: (i, 0)),
        ],
        out_shape=[
            jax.ShapeDtypeStruct((N, DE), jnp.float32),
            jax.ShapeDtypeStruct((N, DE), jnp.float32),
        ],
    )(x, apair, abias)


def _update_body(nf_ref, zn_ref, wt_ref, b_ref, ap_ref, ab_ref,
                 h_ref, s_ref, d_ref):
    acc = zn_ref[0] + zn_ref[1]
    z = acc[:, 0:DE] / (acc[:, DE:AW] + _EPS)
    h = jnp.dot(nf_ref[...], wt_ref[0:D, :], preferred_element_type=jnp.float32)
    h = h + jnp.dot(z, wt_ref[D:D + DE, :], preferred_element_type=jnp.float32)
    h = jnp.maximum(h + b_ref[...], 0.0)
    h_ref[...] = h
    a = (jnp.dot(h, ap_ref[...], preferred_element_type=jnp.float32)
         + ab_ref[...])
    s_ref[...] = jnp.broadcast_to(a[:, 0:1], (a.shape[0], DE))
    d_ref[...] = jnp.broadcast_to(a[:, 1:2], (a.shape[0], DE))


def _update(nf, zn, wt, b, apair, abias):
    return pl.pallas_call(
        _update_body,
        grid=(N // BLK,),
        in_specs=[
            pl.BlockSpec((BLK, D), lambda i: (i, 0)),
            pl.BlockSpec((NC, BLK, AW), lambda i: (0, i, 0)),
            pl.BlockSpec((D + DE, D), lambda i: (0, 0)),
            pl.BlockSpec((1, D), lambda i: (0, 0)),
            pl.BlockSpec((D, 2), lambda i: (0, 0)),
            pl.BlockSpec((1, 2), lambda i: (0, 0)),
        ],
        out_specs=[
            pl.BlockSpec((BLK, D), lambda i: (i, 0)),
            pl.BlockSpec((BLK, DE), lambda i: (i, 0)),
            pl.BlockSpec((BLK, DE), lambda i: (i, 0)),
        ],
        out_shape=[
            jax.ShapeDtypeStruct((N, D), jnp.float32),
            jax.ShapeDtypeStruct((N, DE), jnp.float32),
            jax.ShapeDtypeStruct((N, DE), jnp.float32),
        ],
    )(nf, zn, wt, b, apair, abias)


def kernel(nfeats, efeats, edge_index, W1_w, W1_b, A1_w, A1_b,
           W2_w, W2_b, A2_w, A2_b):
    nf = nfeats.reshape(N, D)
    ef = efeats.reshape(E, DE)
    src = edge_index[0]
    dst = edge_index[1]

    zero1 = jnp.zeros((1,), jnp.float32)
    a1pair = jnp.stack([A1_w[0, :D], A1_w[0, D:]], axis=1)
    a1b = jnp.concatenate([A1_b, zero1]).reshape(1, 2)
    a2pair = jnp.stack([A2_w[0, :D], A2_w[0, D:]], axis=1)
    a2b = jnp.concatenate([A2_b, zero1]).reshape(1, 2)
    w1t = W1_w.T
    w2t = W2_w.T
    b1 = W1_b.reshape(1, D)
    b2 = W2_b.reshape(1, D)

    a1s, a1d = _proj(nf, a1pair, a1b)
    zn1 = _edge_pass(src, dst, a1s, a1d, ef)
    h1, a2s, a2d = _update(nf, zn1, w1t, b1, a2pair, a2b)
    zn2 = _edge_pass(src, dst, a2s, a2d, ef)
    h2, _, _ = _update(h1, zn2, w2t, b2, a2pair, a2b)
    return h2


# speedup vs baseline: 30.1437x; 3.0072x over previous
import jax
import jax.numpy as jnp
from jax import lax
from jax.experimental import pallas as pl
from jax.experimental.pallas import tpu as pltpu
from jax.experimental.pallas import tpu_sc as plsc

N = 10000
E = 320000
D = 128
DE = 16
AW = 2 * DE

NC = 2
NS = 16
NW = NC * NS
EPT = E // NW
CH = 128
NCHUNK = EPT // CH
TAIL = EPT - NCHUNK * CH
NPAIR = NCHUNK // 2
RPT = 624
REM_OFF = NS * RPT
REM = N - REM_OFF

_EPS = 1e-12


def _edge_body(ei_hbm, as_hbm, ad_hbm, ef_hbm, zn_out,
               srcb0, dstb0, sidx0, efb0, s1b0, s2b0, msgb0,
               srcb1, dstb1, sidx1, efb1, s1b1, s2b1, msgb1,
               zrow, zn_sh,
               isem0, gsem0, ssem0, isem1, gsem1, ssem1):
    c = lax.axis_index("c")
    s = lax.axis_index("s")
    wid = s * NC + c
    ebase = wid * EPT

    slots = (
        (srcb0, dstb0, sidx0, efb0, s1b0, s2b0, msgb0, isem0, gsem0, ssem0),
        (srcb1, dstb1, sidx1, efb1, s1b1, s2b1, msgb1, isem1, gsem1, ssem1),
    )

    zeros = jnp.zeros((16,), jnp.float32)
    izeros = jnp.zeros((16,), jnp.int32)

    def _z16(i, _):
        zrow[i, pl.ds(0, 16)] = zeros
        zrow[i, pl.ds(16, 16)] = zeros
        return 0

    lax.fori_loop(0, RPT, _z16, 0)

    rows = pl.ds(s * RPT, RPT)
    rem = pl.ds(REM_OFF, REM)
    pltpu.sync_copy(zrow, zn_sh.at[rows])

    @pl.when(s == NS - 1)
    def _zero_rem():
        pltpu.sync_copy(zrow.at[pl.ds(0, REM)], zn_sh.at[rem])

    plsc.subcore_barrier()

    def _issue_loads(k, slot):
        srcb, dstb, _, efb, _, _, _, isem, _, _ = slots[slot]
        base = ebase + k * CH
        pltpu.async_copy(ei_hbm.at[0, pl.ds(base, CH)], srcb, isem)
        pltpu.async_copy(ei_hbm.at[1, pl.ds(base, CH)], dstb, isem)
        pltpu.async_copy(ef_hbm.at[pl.ds(base, CH)], efb, isem)

    def _wait_loads(slot):
        srcb, dstb, _, efb, _, _, _, isem, _, _ = slots[slot]
        pltpu.make_async_copy(ei_hbm.at[0, pl.ds(0, CH)], srcb, isem).wait()
        pltpu.make_async_copy(ei_hbm.at[1, pl.ds(0, CH)], dstb, isem).wait()
        pltpu.make_async_copy(ef_hbm.at[pl.ds(0, CH)], efb, isem).wait()

    for slot in (0, 1):
        srcb, dstb, sidx, efb, s1b, s2b, msgb, isem, gsem, ssem = slots[slot]

        def _zmsg(i, _, msgb=msgb, sidx=sidx):
            msgb[i, pl.ds(0, 16)] = zeros
            msgb[i, pl.ds(16, 16)] = zeros

            @pl.when(i < CH // 16)
            def _zi():
                sidx[pl.ds(i * 16, 16)] = izeros

            return 0

        lax.fori_loop(0, CH, _zmsg, 0)
        pltpu.async_copy(msgb, zn_sh.at[sidx], ssem, add=True)
        _issue_loads(slot, slot)

    def _process(k, slot):
        srcb, dstb, sidx, efb, s1b, s2b, msgb, isem, gsem, ssem = slots[slot]
        _wait_loads(slot)
        g1 = pltpu.async_copy(as_hbm.at[srcb], s1b, gsem)
        g2 = pltpu.async_copy(ad_hbm.at[dstb], s2b, gsem)
        g1.wait()
        g2.wait()
        pltpu.make_async_copy(msgb, zn_sh.at[sidx], ssem).wait()

        def _sidx(g, _):
            sidx[pl.ds(g * 16, 16)] = dstb[pl.ds(g * 16, 16)]
            return 0

        lax.fori_loop(0, CH // 16, _sidx, 0)

        @plsc.parallel_loop(0, CH, unroll=4)
        def _edge(e):
            w = jnp.exp(jnp.maximum(s1b[e, :] + s2b[e, :], 0.0))
            msgb[e, pl.ds(0, DE)] = w * efb[e, :]
            msgb[e, pl.ds(DE, DE)] = w

        pltpu.async_copy(msgb, zn_sh.at[sidx], ssem, add=True)
        _issue_loads(jnp.minimum(k + 2, NCHUNK - 1), slot)

    def _pair(kk, _):
        _process(2 * kk, 0)
        _process(2 * kk + 1, 1)
        return 0

    lax.fori_loop(0, NPAIR, _pair, 0)

    for slot in (0, 1):
        srcb, dstb, sidx, efb, s1b, s2b, msgb, isem, gsem, ssem = slots[slot]
        _wait_loads(slot)
        pltpu.make_async_copy(msgb, zn_sh.at[sidx], ssem).wait()

    tb = pl.ds(0, TAIL)
    tbase = ebase + NCHUNK * CH
    srcb, dstb, sidx, efb, s1b, s2b, msgb, isem, gsem, ssem = slots[0]
    pltpu.sync_copy(ei_hbm.at[0, pl.ds(tbase, TAIL)], srcb.at[tb])
    pltpu.sync_copy(ei_hbm.at[1, pl.ds(tbase, TAIL)], dstb.at[tb])
    pltpu.sync_copy(ef_hbm.at[pl.ds(tbase, TAIL)], efb.at[tb])
    pltpu.async_copy(as_hbm.at[srcb.at[tb]], s1b.at[tb], gsem).wait()
    pltpu.async_copy(ad_hbm.at[dstb.at[tb]], s2b.at[tb], gsem).wait()
    sidx[tb] = dstb[tb]

    def _tedge(e, _):
        w = jnp.exp(jnp.maximum(s1b[e, :] + s2b[e, :], 0.0))
        msgb[e, pl.ds(0, DE)] = w * efb[e, :]
        msgb[e, pl.ds(DE, DE)] = w
        return 0

    lax.fori_loop(0, TAIL, _tedge, 0)
    pltpu.sync_copy(msgb.at[tb], zn_sh.at[sidx.at[tb]], add=True)

    plsc.subcore_barrier()

    pltpu.sync_copy(zn_sh.at[rows], zrow)
    pltpu.sync_copy(zrow, zn_out.at[c, rows])

    @pl.when(s == NS - 1)
    def _flush_rem():
        pltpu.sync_copy(zn_sh.at[rem], zrow.at[pl.ds(0, REM)])
        pltpu.sync_copy(zrow.at[pl.ds(0, REM)], zn_out.at[c, rem])


def _slot_scratch():
    return [
        pltpu.VMEM((CH,), jnp.int32),
        pltpu.VMEM((CH,), jnp.int32),
        pltpu.VMEM((CH,), jnp.int32),
        pltpu.VMEM((CH, DE), jnp.float32),
        pltpu.VMEM((CH, DE), jnp.float32),
        pltpu.VMEM((CH, DE), jnp.float32),
        pltpu.VMEM((CH, AW), jnp.float32),
    ]


_edge_pass = pl.kernel(
    _edge_body,
    out_type=jax.ShapeDtypeStruct((NC, N, AW), jnp.float32),
    mesh=plsc.VectorSubcoreMesh(core_axis_name="c", subcore_axis_name="s"),
    compiler_params=pltpu.CompilerParams(use_tc_tiling_on_sc=False),
    scratch_types=(
        _slot_scratch() + _slot_scratch() + [
            pltpu.VMEM((RPT, AW), jnp.float32),
            pltpu.VMEM_SHARED((N, AW), jnp.float32),
            pltpu.SemaphoreType.DMA,
            pltpu.SemaphoreType.DMA,
            pltpu.SemaphoreType.DMA,
            pltpu.SemaphoreType.DMA,
            pltpu.SemaphoreType.DMA,
            pltpu.SemaphoreType.DMA,
        ]
    ),
)


BLK = 2000


def _proj_body(x_ref, ap_ref, ab_ref, s_ref, d_ref):
    a = (jnp.dot(x_ref[...], ap_ref[...], preferred_element_type=jnp.float32)
         + ab_ref[...])
    s_ref[...] = jnp.broadcast_to(a[:, 0:1], (a.shape[0], DE))
    d_ref[...] = jnp.broadcast_to(a[:, 1:2], (a.shape[0], DE))


def _proj(x, apair, abias):
    return pl.pallas_call(
        _proj_body,
        grid=(N // BLK,),
        in_specs=[
            pl.BlockSpec((BLK, D), lambda i: (i, 0)),
            pl.BlockSpec((D, 2), lambda i: (0, 0)),
            pl.BlockSpec((1, 2), lambda i: (0, 0)),
        ],
        out_specs=[
            pl.BlockSpec((BLK, DE), lambda i: (i, 0)),
            pl.BlockSpec((BLK, DE), lambda i: (i, 0)),
        ],
        out_shape=[
            jax.ShapeDtypeStruct((N, DE), jnp.float32),
            jax.ShapeDtypeStruct((N, DE), jnp.float32),
        ],
    )(x, apair, abias)


def _update_body(nf_ref, zn_ref, wt_ref, b_ref, ap_ref, ab_ref,
                 h_ref, s_ref, d_ref):
    acc = zn_ref[0] + zn_ref[1]
    z = acc[:, 0:DE] / (acc[:, DE:AW] + _EPS)
    h = jnp.dot(nf_ref[...], wt_ref[0:D, :], preferred_element_type=jnp.float32)
    h = h + jnp.dot(z, wt_ref[D:D + DE, :], preferred_element_type=jnp.float32)
    h = jnp.maximum(h + b_ref[...], 0.0)
    h_ref[...] = h
    a = (jnp.dot(h, ap_ref[...], preferred_element_type=jnp.float32)
         + ab_ref[...])
    s_ref[...] = jnp.broadcast_to(a[:, 0:1], (a.shape[0], DE))
    d_ref[...] = jnp.broadcast_to(a[:, 1:2], (a.shape[0], DE))


def _update(nf, zn, wt, b, apair, abias):
    return pl.pallas_call(
        _update_body,
        grid=(N // BLK,),
        in_specs=[
            pl.BlockSpec((BLK, D), lambda i: (i, 0)),
            pl.BlockSpec((NC, BLK, AW), lambda i: (0, i, 0)),
            pl.BlockSpec((D + DE, D), lambda i: (0, 0)),
            pl.BlockSpec((1, D), lambda i: (0, 0)),
            pl.BlockSpec((D, 2), lambda i: (0, 0)),
            pl.BlockSpec((1, 2), lambda i: (0, 0)),
        ],
        out_specs=[
            pl.BlockSpec((BLK, D), lambda i: (i, 0)),
            pl.BlockSpec((BLK, DE), lambda i: (i, 0)),
            pl.BlockSpec((BLK, DE), lambda i: (i, 0)),
        ],
        out_shape=[
            jax.ShapeDtypeStruct((N, D), jnp.float32),
            jax.ShapeDtypeStruct((N, DE), jnp.float32),
            jax.ShapeDtypeStruct((N, DE), jnp.float32),
        ],
    )(nf, zn, wt, b, apair, abias)


def kernel(nfeats, efeats, edge_index, W1_w, W1_b, A1_w, A1_b,
           W2_w, W2_b, A2_w, A2_b):
    nf = nfeats.reshape(N, D)
    ef = efeats.reshape(E, DE)

    zero1 = jnp.zeros((1,), jnp.float32)
    a1pair = jnp.stack([A1_w[0, :D], A1_w[0, D:]], axis=1)
    a1b = jnp.concatenate([A1_b, zero1]).reshape(1, 2)
    a2pair = jnp.stack([A2_w[0, :D], A2_w[0, D:]], axis=1)
    a2b = jnp.concatenate([A2_b, zero1]).reshape(1, 2)
    w1t = W1_w.T
    w2t = W2_w.T
    b1 = W1_b.reshape(1, D)
    b2 = W2_b.reshape(1, D)

    a1s, a1d = _proj(nf, a1pair, a1b)
    zn1 = _edge_pass(edge_index, a1s, a1d, ef)
    h1, a2s, a2d = _update(nf, zn1, w1t, b1, a2pair, a2b)
    zn2 = _edge_pass(edge_index, a2s, a2d, ef)
    h2, _, _ = _update(h1, zn2, w2t, b2, a2pair, a2b)
    return h2


# speedup vs baseline: 35.3273x; 1.1720x over previous
import jax
import jax.numpy as jnp
from jax import lax
from jax.experimental import pallas as pl
from jax.experimental.pallas import tpu as pltpu
from jax.experimental.pallas import tpu_sc as plsc

N = 10000
E = 320000
D = 128
DE = 16
AW = 2 * DE

NC = 2
NS = 16
NW = NC * NS
EPT = E // NW
CH = 128
NCHUNK = EPT // CH
TAIL = EPT - NCHUNK * CH
SD = 4
LOOPC = NCHUNK - 2
NQUAD = LOOPC // SD
RPT = 624
REM_OFF = NS * RPT
REM = N - REM_OFF

_EPS = 1e-12


def _edge_body(ei_hbm, as_hbm, ad_hbm, ef_hbm, zn_out, *refs):
    slots = tuple(refs[i * 7:(i + 1) * 7] for i in range(SD))
    zrow, zn_sh = refs[7 * SD], refs[7 * SD + 1]
    isems = refs[7 * SD + 2:7 * SD + 2 + SD]
    gsems = refs[7 * SD + 2 + SD:7 * SD + 2 + 2 * SD]
    ssems = refs[7 * SD + 2 + 2 * SD:7 * SD + 2 + 3 * SD]

    c = lax.axis_index("c")
    s = lax.axis_index("s")
    wid = s * NC + c
    ebase = wid * EPT

    zeros = jnp.zeros((16,), jnp.float32)
    izeros = jnp.zeros((16,), jnp.int32)

    def _z16(i, _):
        zrow[i, pl.ds(0, 16)] = zeros
        zrow[i, pl.ds(16, 16)] = zeros
        return 0

    lax.fori_loop(0, RPT, _z16, 0)

    rows = pl.ds(s * RPT, RPT)
    rem = pl.ds(REM_OFF, REM)
    pltpu.sync_copy(zrow, zn_sh.at[rows])

    @pl.when(s == NS - 1)
    def _zero_rem():
        pltpu.sync_copy(zrow.at[pl.ds(0, REM)], zn_sh.at[rem])

    plsc.subcore_barrier()

    def _issue_loads(k, slot):
        srcb, dstb, _, efb, _, _, _ = slots[slot]
        base = ebase + k * CH
        pltpu.async_copy(ei_hbm.at[0, pl.ds(base, CH)], srcb, isems[slot])
        pltpu.async_copy(ei_hbm.at[1, pl.ds(base, CH)], dstb, isems[slot])
        pltpu.async_copy(ef_hbm.at[pl.ds(base, CH), 0], efb, isems[slot])

    def _wait_loads(slot):
        srcb, dstb, _, efb, _, _, _ = slots[slot]
        isem = isems[slot]
        pltpu.make_async_copy(ei_hbm.at[0, pl.ds(0, CH)], srcb, isem).wait()
        pltpu.make_async_copy(ei_hbm.at[1, pl.ds(0, CH)], dstb, isem).wait()
        pltpu.make_async_copy(ef_hbm.at[pl.ds(0, CH), 0], efb, isem).wait()

    def _issue_gathers(slot):
        srcb, dstb, _, _, s1b, s2b, _ = slots[slot]
        pltpu.async_copy(as_hbm.at[srcb], s1b, gsems[slot])
        pltpu.async_copy(ad_hbm.at[dstb], s2b, gsems[slot])

    def _wait_gathers(slot):
        srcb, dstb, _, _, s1b, s2b, _ = slots[slot]
        gsem = gsems[slot]
        pltpu.make_async_copy(as_hbm.at[srcb], s1b, gsem).wait()
        pltpu.make_async_copy(ad_hbm.at[dstb], s2b, gsem).wait()

    def _wait_scatter(slot):
        _, _, sidx, _, _, _, msgb = slots[slot]
        pltpu.make_async_copy(msgb, zn_sh.at[sidx], ssems[slot]).wait()

    def _compute_scatter(slot):
        _, dstb, sidx, efb, s1b, s2b, msgb = slots[slot]

        def _sidx(g, _):
            sidx[pl.ds(g * 16, 16)] = dstb[pl.ds(g * 16, 16)]
            return 0

        lax.fori_loop(0, CH // 16, _sidx, 0)

        @plsc.parallel_loop(0, CH, unroll=4)
        def _edge(e):
            w = jnp.exp(jnp.maximum(s1b[e, :] + s2b[e, :], 0.0))
            msgb[e, pl.ds(0, DE)] = w * efb[e, :]
            msgb[e, pl.ds(DE, DE)] = w

        pltpu.async_copy(msgb, zn_sh.at[sidx], ssems[slot], add=True)

    for slot in range(SD):
        _, _, sidx, _, _, _, msgb = slots[slot]

        def _zmsg(i, _, msgb=msgb, sidx=sidx):
            msgb[i, pl.ds(0, 16)] = zeros
            msgb[i, pl.ds(16, 16)] = zeros

            @pl.when(i < CH // 16)
            def _zi():
                sidx[pl.ds(i * 16, 16)] = izeros

            return 0

        lax.fori_loop(0, CH, _zmsg, 0)
        pltpu.async_copy(msgb, zn_sh.at[sidx], ssems[slot], add=True)

    _issue_loads(0, 0)
    _issue_loads(1, 1)
    _wait_loads(0)
    _issue_gathers(0)

    def _step(k, slot):
        _wait_loads((slot + 1) % SD)
        _issue_gathers((slot + 1) % SD)
        _issue_loads(k + 2, (slot + 2) % SD)
        _wait_gathers(slot)
        _wait_scatter(slot)
        _compute_scatter(slot)

    def _quad(jj, _):
        for u in range(SD):
            _step(SD * jj + u, u)
        return 0

    lax.fori_loop(0, NQUAD, _quad, 0)

    _wait_loads(1)
    _issue_gathers(1)
    _wait_gathers(0)
    _wait_scatter(0)
    _compute_scatter(0)
    _wait_gathers(1)
    _wait_scatter(1)
    _compute_scatter(1)
    for slot in range(SD):
        _wait_scatter(slot)

    tb = pl.ds(0, TAIL)
    tbase = ebase + NCHUNK * CH
    srcb, dstb, sidx, efb, s1b, s2b, msgb = slots[2]
    pltpu.sync_copy(ei_hbm.at[0, pl.ds(tbase, TAIL)], srcb.at[tb])
    pltpu.sync_copy(ei_hbm.at[1, pl.ds(tbase, TAIL)], dstb.at[tb])
    pltpu.sync_copy(ef_hbm.at[pl.ds(tbase, TAIL), 0], efb.at[tb])
    pltpu.async_copy(as_hbm.at[srcb.at[tb]], s1b.at[tb], gsems[2]).wait()
    pltpu.async_copy(ad_hbm.at[dstb.at[tb]], s2b.at[tb], gsems[2]).wait()
    sidx[tb] = dstb[tb]

    def _tedge(e, _):
        w = jnp.exp(jnp.maximum(s1b[e, :] + s2b[e, :], 0.0))
        msgb[e, pl.ds(0, DE)] = w * efb[e, :]
        msgb[e, pl.ds(DE, DE)] = w
        return 0

    lax.fori_loop(0, TAIL, _tedge, 0)
    pltpu.sync_copy(msgb.at[tb], zn_sh.at[sidx.at[tb]], add=True)

    plsc.subcore_barrier()

    pltpu.sync_copy(zn_sh.at[rows], zrow)
    pltpu.sync_copy(zrow, zn_out.at[c, rows])

    @pl.when(s == NS - 1)
    def _flush_rem():
        pltpu.sync_copy(zn_sh.at[rem], zrow.at[pl.ds(0, REM)])
        pltpu.sync_copy(zrow.at[pl.ds(0, REM)], zn_out.at[c, rem])


def _slot_scratch():
    return [
        pltpu.VMEM((CH,), jnp.int32),
        pltpu.VMEM((CH,), jnp.int32),
        pltpu.VMEM((CH,), jnp.int32),
        pltpu.VMEM((CH, DE), jnp.float32),
        pltpu.VMEM((CH, DE), jnp.float32),
        pltpu.VMEM((CH, DE), jnp.float32),
        pltpu.VMEM((CH, AW), jnp.float32),
    ]


_edge_pass = pl.kernel(
    _edge_body,
    out_type=jax.ShapeDtypeStruct((NC, N, AW), jnp.float32),
    mesh=plsc.VectorSubcoreMesh(core_axis_name="c", subcore_axis_name="s"),
    compiler_params=pltpu.CompilerParams(use_tc_tiling_on_sc=False),
    scratch_types=(
        _slot_scratch() + _slot_scratch() + _slot_scratch() + _slot_scratch()
        + [
            pltpu.VMEM((RPT, AW), jnp.float32),
            pltpu.VMEM_SHARED((N, AW), jnp.float32),
        ]
        + [pltpu.SemaphoreType.DMA] * (3 * SD)
    ),
)


BLK = 2000


def _proj_body(x_ref, ap_ref, ab_ref, s_ref, d_ref):
    a = (jnp.dot(x_ref[:, 0, :], ap_ref[...],
                 preferred_element_type=jnp.float32) + ab_ref[...])
    s_ref[...] = jnp.broadcast_to(a[:, 0:1], (BLK, DE))
    d_ref[...] = jnp.broadcast_to(a[:, 1:2], (BLK, DE))


def _proj(x, apair, abias):
    return pl.pallas_call(
        _proj_body,
        grid=(N // BLK,),
        in_specs=[
            pl.BlockSpec((BLK, 1, D), lambda i: (i, 0, 0)),
            pl.BlockSpec((D, 2), lambda i: (0, 0)),
            pl.BlockSpec((1, 2), lambda i: (0, 0)),
        ],
        out_specs=[
            pl.BlockSpec((BLK, DE), lambda i: (i, 0)),
            pl.BlockSpec((BLK, DE), lambda i: (i, 0)),
        ],
        out_shape=[
            jax.ShapeDtypeStruct((N, DE), jnp.float32),
            jax.ShapeDtypeStruct((N, DE), jnp.float32),
        ],
    )(x, apair, abias)


def _update_body(nf_ref, zn_ref, wt_ref, b_ref, ap_ref, ab_ref,
                 h_ref, s_ref, d_ref):
    acc = zn_ref[0] + zn_ref[1]
    z = acc[:, 0:DE] / (acc[:, DE:AW] + _EPS)
    h = jnp.dot(nf_ref[:, 0, :], wt_ref[0:D, :],
                preferred_element_type=jnp.float32)
    h = h + jnp.dot(z, wt_ref[D:D + DE, :], preferred_element_type=jnp.float32)
    h = jnp.maximum(h + b_ref[...], 0.0)
    h_ref[:, 0, :] = h
    a = (jnp.dot(h, ap_ref[...], preferred_element_type=jnp.float32)
         + ab_ref[...])
    s_ref[...] = jnp.broadcast_to(a[:, 0:1], (BLK, DE))
    d_ref[...] = jnp.broadcast_to(a[:, 1:2], (BLK, DE))


def _update(nf, zn, wt, b, apair, abias):
    return pl.pallas_call(
        _update_body,
        grid=(N // BLK,),
        in_specs=[
            pl.BlockSpec((BLK, 1, D), lambda i: (i, 0, 0)),
            pl.BlockSpec((NC, BLK, AW), lambda i: (0, i, 0)),
            pl.BlockSpec((D + DE, D), lambda i: (0, 0)),
            pl.BlockSpec((1, D), lambda i: (0, 0)),
            pl.BlockSpec((D, 2), lambda i: (0, 0)),
            pl.BlockSpec((1, 2), lambda i: (0, 0)),
        ],
        out_specs=[
            pl.BlockSpec((BLK, 1, D), lambda i: (i, 0, 0)),
            pl.BlockSpec((BLK, DE), lambda i: (i, 0)),
            pl.BlockSpec((BLK, DE), lambda i: (i, 0)),
        ],
        out_shape=[
            jax.ShapeDtypeStruct((N, 1, D), jnp.float32),
            jax.ShapeDtypeStruct((N, DE), jnp.float32),
            jax.ShapeDtypeStruct((N, DE), jnp.float32),
        ],
    )(nf, zn, wt, b, apair, abias)


def kernel(nfeats, efeats, edge_index, W1_w, W1_b, A1_w, A1_b,
           W2_w, W2_b, A2_w, A2_b):
    zero1 = jnp.zeros((1,), jnp.float32)
    a1pair = jnp.stack([A1_w[0, :D], A1_w[0, D:]], axis=1)
    a1b = jnp.concatenate([A1_b, zero1]).reshape(1, 2)
    a2pair = jnp.stack([A2_w[0, :D], A2_w[0, D:]], axis=1)
    a2b = jnp.concatenate([A2_b, zero1]).reshape(1, 2)
    w1t = W1_w.T
    w2t = W2_w.T
    b1 = W1_b.reshape(1, D)
    b2 = W2_b.reshape(1, D)

    a1s, a1d = _proj(nfeats, a1pair, a1b)
    zn1 = _edge_pass(edge_index, a1s, a1d, efeats)
    h1, a2s, a2d = _update(nfeats, zn1, w1t, b1, a2pair, a2b)
    zn2 = _edge_pass(edge_index, a2s, a2d, efeats)
    h2, _, _ = _update(h1, zn2, w2t, b2, a2pair, a2b)
    return h2.reshape(N, D)


# speedup vs baseline: 35.4409x; 1.0032x over previous
import jax
import jax.numpy as jnp
from jax import lax
from jax.experimental import pallas as pl
from jax.experimental.pallas import tpu as pltpu
from jax.experimental.pallas import tpu_sc as plsc

N = 10000
E = 320000
D = 128
DE = 16
AW = 2 * DE

NC = 2
NS = 16
NW = NC * NS
EPT = E // NW
CH = 128
NCHUNK = EPT // CH
TAIL = EPT - NCHUNK * CH
SD = 4
LOOPC = NCHUNK - 6
NQUAD = LOOPC // SD
RPT = 624
REM_OFF = NS * RPT
REM = N - REM_OFF

_EPS = 1e-12


def _edge_body(ei_hbm, as_hbm, ad_hbm, ef_hbm, zn_out, *refs):
    slots = tuple(refs[i * 7:(i + 1) * 7] for i in range(SD))
    zrow, zn_sh = refs[7 * SD], refs[7 * SD + 1]
    isems = refs[7 * SD + 2:7 * SD + 2 + SD]
    gsems = refs[7 * SD + 2 + SD:7 * SD + 2 + 2 * SD]
    ssems = refs[7 * SD + 2 + 2 * SD:7 * SD + 2 + 3 * SD]

    c = lax.axis_index("c")
    s = lax.axis_index("s")
    wid = s * NC + c
    ebase = wid * EPT

    zeros = jnp.zeros((16,), jnp.float32)
    izeros = jnp.zeros((16,), jnp.int32)

    def _z16(i, _):
        zrow[i, pl.ds(0, 16)] = zeros
        zrow[i, pl.ds(16, 16)] = zeros
        return 0

    lax.fori_loop(0, RPT, _z16, 0)

    rows = pl.ds(s * RPT, RPT)
    rem = pl.ds(REM_OFF, REM)
    pltpu.sync_copy(zrow, zn_sh.at[rows])

    @pl.when(s == NS - 1)
    def _zero_rem():
        pltpu.sync_copy(zrow.at[pl.ds(0, REM)], zn_sh.at[rem])

    plsc.subcore_barrier()

    def _issue_loads(k, slot):
        srcb, dstb, _, efb, _, _, _ = slots[slot]
        base = ebase + k * CH
        pltpu.async_copy(ei_hbm.at[0, pl.ds(base, CH)], srcb, isems[slot])
        pltpu.async_copy(ei_hbm.at[1, pl.ds(base, CH)], dstb, isems[slot])
        pltpu.async_copy(ef_hbm.at[pl.ds(base, CH), 0], efb, isems[slot])

    def _wait_loads(slot):
        srcb, dstb, _, efb, _, _, _ = slots[slot]
        isem = isems[slot]
        pltpu.make_async_copy(ei_hbm.at[0, pl.ds(0, CH)], srcb, isem).wait()
        pltpu.make_async_copy(ei_hbm.at[1, pl.ds(0, CH)], dstb, isem).wait()
        pltpu.make_async_copy(ef_hbm.at[pl.ds(0, CH), 0], efb, isem).wait()

    def _issue_gathers(slot):
        srcb, dstb, _, _, s1b, s2b, _ = slots[slot]
        pltpu.async_copy(as_hbm.at[srcb], s1b, gsems[slot])
        pltpu.async_copy(ad_hbm.at[dstb], s2b, gsems[slot])

    def _wait_gathers(slot):
        srcb, dstb, _, _, s1b, s2b, _ = slots[slot]
        gsem = gsems[slot]
        pltpu.make_async_copy(as_hbm.at[srcb], s1b, gsem).wait()
        pltpu.make_async_copy(ad_hbm.at[dstb], s2b, gsem).wait()

    def _wait_scatter(slot):
        _, _, sidx, _, _, _, msgb = slots[slot]
        pltpu.make_async_copy(msgb, zn_sh.at[sidx], ssems[slot]).wait()

    def _compute_scatter(slot):
        _, dstb, sidx, efb, s1b, s2b, msgb = slots[slot]

        def _sidx(g, _):
            sidx[pl.ds(g * 16, 16)] = dstb[pl.ds(g * 16, 16)]
            return 0

        lax.fori_loop(0, CH // 16, _sidx, 0)

        @plsc.parallel_loop(0, CH, unroll=4)
        def _edge(e):
            w = jnp.exp(jnp.maximum(s1b[e, :] + s2b[e, :], 0.0))
            msgb[e, pl.ds(0, DE)] = w * efb[e, :]
            msgb[e, pl.ds(DE, DE)] = w

        pltpu.async_copy(msgb, zn_sh.at[sidx], ssems[slot], add=True)

    for slot in range(SD):
        _, _, sidx, _, _, _, msgb = slots[slot]

        def _zmsg(i, _, msgb=msgb, sidx=sidx):
            msgb[i, pl.ds(0, 16)] = zeros
            msgb[i, pl.ds(16, 16)] = zeros

            @pl.when(i < CH // 16)
            def _zi():
                sidx[pl.ds(i * 16, 16)] = izeros

            return 0

        lax.fori_loop(0, CH, _zmsg, 0)
        pltpu.async_copy(msgb, zn_sh.at[sidx], ssems[slot], add=True)

    _issue_loads(0, 0)
    _issue_loads(1, 1)
    _issue_loads(2, 2)
    _wait_loads(0)
    _issue_gathers(0)
    _wait_loads(1)
    _issue_gathers(1)

    def _step(k, slot):
        _wait_gathers(slot)
        _wait_loads((slot + 2) % SD)
        _issue_gathers((slot + 2) % SD)
        _issue_loads(k + 3, (slot + 3) % SD)
        _wait_scatter(slot)
        _compute_scatter(slot)

    def _quad(jj, _):
        for u in range(SD):
            _step(SD * jj + u, u)
        return 0

    lax.fori_loop(0, NQUAD, _quad, 0)

    for k in range(SD * NQUAD, NCHUNK):
        slot = k % SD
        _wait_gathers(slot)
        if k + 2 < NCHUNK:
            _wait_loads((slot + 2) % SD)
            _issue_gathers((slot + 2) % SD)
        if k + 3 < NCHUNK:
            _issue_loads(k + 3, (slot + 3) % SD)
        _wait_scatter(slot)
        _compute_scatter(slot)
    for slot in range(SD):
        _wait_scatter(slot)

    tb = pl.ds(0, TAIL)
    tbase = ebase + NCHUNK * CH
    srcb, dstb, sidx, efb, s1b, s2b, msgb = slots[2]
    pltpu.sync_copy(ei_hbm.at[0, pl.ds(tbase, TAIL)], srcb.at[tb])
    pltpu.sync_copy(ei_hbm.at[1, pl.ds(tbase, TAIL)], dstb.at[tb])
    pltpu.sync_copy(ef_hbm.at[pl.ds(tbase, TAIL), 0], efb.at[tb])
    pltpu.async_copy(as_hbm.at[srcb.at[tb]], s1b.at[tb], gsems[2]).wait()
    pltpu.async_copy(ad_hbm.at[dstb.at[tb]], s2b.at[tb], gsems[2]).wait()
    sidx[tb] = dstb[tb]

    def _tedge(e, _):
        w = jnp.exp(jnp.maximum(s1b[e, :] + s2b[e, :], 0.0))
        msgb[e, pl.ds(0, DE)] = w * efb[e, :]
        msgb[e, pl.ds(DE, DE)] = w
        return 0

    lax.fori_loop(0, TAIL, _tedge, 0)
    pltpu.sync_copy(msgb.at[tb], zn_sh.at[sidx.at[tb]], add=True)

    plsc.subcore_barrier()

    pltpu.sync_copy(zn_sh.at[rows], zrow)
    pltpu.sync_copy(zrow, zn_out.at[c, rows])

    @pl.when(s == NS - 1)
    def _flush_rem():
        pltpu.sync_copy(zn_sh.at[rem], zrow.at[pl.ds(0, REM)])
        pltpu.sync_copy(zrow.at[pl.ds(0, REM)], zn_out.at[c, rem])


def _slot_scratch():
    return [
        pltpu.VMEM((CH,), jnp.int32),
        pltpu.VMEM((CH,), jnp.int32),
        pltpu.VMEM((CH,), jnp.int32),
        pltpu.VMEM((CH, DE), jnp.float32),
        pltpu.VMEM((CH, DE), jnp.float32),
        pltpu.VMEM((CH, DE), jnp.float32),
        pltpu.VMEM((CH, AW), jnp.float32),
    ]


_edge_pass = pl.kernel(
    _edge_body,
    out_type=jax.ShapeDtypeStruct((NC, N, AW), jnp.float32),
    mesh=plsc.VectorSubcoreMesh(core_axis_name="c", subcore_axis_name="s"),
    compiler_params=pltpu.CompilerParams(use_tc_tiling_on_sc=False),
    scratch_types=(
        _slot_scratch() + _slot_scratch() + _slot_scratch() + _slot_scratch()
        + [
            pltpu.VMEM((RPT, AW), jnp.float32),
            pltpu.VMEM_SHARED((N, AW), jnp.float32),
        ]
        + [pltpu.SemaphoreType.DMA] * (3 * SD)
    ),
)


BLK = 2000


def _proj_body(x_ref, ap_ref, ab_ref, s_ref, d_ref):
    a = (jnp.dot(x_ref[:, 0, :], ap_ref[...],
                 preferred_element_type=jnp.float32) + ab_ref[...])
    s_ref[...] = jnp.broadcast_to(a[:, 0:1], (BLK, DE))
    d_ref[...] = jnp.broadcast_to(a[:, 1:2], (BLK, DE))


def _proj(x, apair, abias):
    return pl.pallas_call(
        _proj_body,
        grid=(N // BLK,),
        in_specs=[
            pl.BlockSpec((BLK, 1, D), lambda i: (i, 0, 0)),
            pl.BlockSpec((D, 2), lambda i: (0, 0)),
            pl.BlockSpec((1, 2), lambda i: (0, 0)),
        ],
        out_specs=[
            pl.BlockSpec((BLK, DE), lambda i: (i, 0)),
            pl.BlockSpec((BLK, DE), lambda i: (i, 0)),
        ],
        out_shape=[
            jax.ShapeDtypeStruct((N, DE), jnp.float32),
            jax.ShapeDtypeStruct((N, DE), jnp.float32),
        ],
    )(x, apair, abias)


def _update_body(nf_ref, zn_ref, wt_ref, b_ref, ap_ref, ab_ref,
                 h_ref, s_ref, d_ref):
    acc = zn_ref[0] + zn_ref[1]
    z = acc[:, 0:DE] / (acc[:, DE:AW] + _EPS)
    h = jnp.dot(nf_ref[:, 0, :], wt_ref[0:D, :],
                preferred_element_type=jnp.float32)
    h = h + jnp.dot(z, wt_ref[D:D + DE, :], preferred_element_type=jnp.float32)
    h = jnp.maximum(h + b_ref[...], 0.0)
    h_ref[:, 0, :] = h
    a = (jnp.dot(h, ap_ref[...], preferred_element_type=jnp.float32)
         + ab_ref[...])
    s_ref[...] = jnp.broadcast_to(a[:, 0:1], (BLK, DE))
    d_ref[...] = jnp.broadcast_to(a[:, 1:2], (BLK, DE))


def _update(nf, zn, wt, b, apair, abias):
    return pl.pallas_call(
        _update_body,
        grid=(N // BLK,),
        in_specs=[
            pl.BlockSpec((BLK, 1, D), lambda i: (i, 0, 0)),
            pl.BlockSpec((NC, BLK, AW), lambda i: (0, i, 0)),
            pl.BlockSpec((D + DE, D), lambda i: (0, 0)),
            pl.BlockSpec((1, D), lambda i: (0, 0)),
            pl.BlockSpec((D, 2), lambda i: (0, 0)),
            pl.BlockSpec((1, 2), lambda i: (0, 0)),
        ],
        out_specs=[
            pl.BlockSpec((BLK, 1, D), lambda i: (i, 0, 0)),
            pl.BlockSpec((BLK, DE), lambda i: (i, 0)),
            pl.BlockSpec((BLK, DE), lambda i: (i, 0)),
        ],
        out_shape=[
            jax.ShapeDtypeStruct((N, 1, D), jnp.float32),
            jax.ShapeDtypeStruct((N, DE), jnp.float32),
            jax.ShapeDtypeStruct((N, DE), jnp.float32),
        ],
    )(nf, zn, wt, b, apair, abias)


def kernel(nfeats, efeats, edge_index, W1_w, W1_b, A1_w, A1_b,
           W2_w, W2_b, A2_w, A2_b):
    zero1 = jnp.zeros((1,), jnp.float32)
    a1pair = jnp.stack([A1_w[0, :D], A1_w[0, D:]], axis=1)
    a1b = jnp.concatenate([A1_b, zero1]).reshape(1, 2)
    a2pair = jnp.stack([A2_w[0, :D], A2_w[0, D:]], axis=1)
    a2b = jnp.concatenate([A2_b, zero1]).reshape(1, 2)
    w1t = W1_w.T
    w2t = W2_w.T
    b1 = W1_b.reshape(1, D)
    b2 = W2_b.reshape(1, D)

    a1s, a1d = _proj(nfeats, a1pair, a1b)
    zn1 = _edge_pass(edge_index, a1s, a1d, efeats)
    h1, a2s, a2d = _update(nfeats, zn1, w1t, b1, a2pair, a2b)
    zn2 = _edge_pass(edge_index, a2s, a2d, efeats)
    h2, _, _ = _update(h1, zn2, w2t, b2, a2pair, a2b)
    return h2.reshape(N, D)
